# Initial kernel scaffold; baseline (speedup 1.0000x reference)
import jax
import jax.numpy as jnp
from jax.experimental import pallas as pl


def kernel(x_i, fg_sdf, bg_sdf):
    raise NotImplementedError("write your pallas kernel here")


# speedup vs baseline: 22.2171x; 22.2171x over previous
import functools

import jax
import jax.numpy as jnp
from jax import lax
from jax.experimental import pallas as pl
from jax.experimental.pallas import tpu as pltpu
from jax.experimental.pallas import tpu_sc as plsc

N_PTS = 2_000_000
FG_RES = 128
BG_RES = 256
FG_SIZE = FG_RES ** 3
TAB_SIZE = FG_RES ** 3 + BG_RES ** 3

NC = 2
NS = 16
NW = NC * NS
LANES = 16

C = 2000
G = C // LANES
NCHUNK = N_PTS // C

_mesh = plsc.VectorSubcoreMesh(core_axis_name="c", subcore_axis_name="s")


def _axis(c, inv_ext, szm1_f, szm1_i):
    p = (c * inv_ext + 1.0) * 0.5 * szm1_f
    p = jnp.minimum(jnp.maximum(p, 0.0), szm1_f)
    i0 = p.astype(jnp.int32)
    w = p - i0.astype(jnp.float32)
    i1 = jnp.minimum(i0 + 1, szm1_i)
    return i0, i1, w


@functools.partial(
    pl.kernel,
    mesh=_mesh,
    out_type=jax.ShapeDtypeStruct((N_PTS,), jnp.float32),
    scratch_types=[
        pltpu.VMEM((C,), jnp.float32),
        pltpu.VMEM((C,), jnp.float32),
        pltpu.VMEM((C,), jnp.float32),
        pltpu.VMEM((8 * C,), jnp.int32),
        pltpu.VMEM((8 * C,), jnp.float32),
        pltpu.VMEM((C,), jnp.float32),
        pltpu.VMEM((C,), jnp.float32),
        pltpu.VMEM((C,), jnp.float32),
        pltpu.VMEM((C,), jnp.float32),
        pltpu.VMEM((C,), jnp.float32),
        pltpu.SemaphoreType.DMA,
    ],
)
def _sdf_kernel(cw_hbm, ch_hbm, cd_hbm, tab_hbm, out_hbm,
                cw_v, ch_v, cd_v, idx_v, g_v, wx_v, wy_v, wz_v, m_v, out_v,
                sem):
    wid = lax.axis_index("s") * NC + lax.axis_index("c")
    nchunks_w = (NCHUNK - wid + NW - 1) // NW

    def chunk_body(i, carry):
        base = (wid + i * NW) * C
        d1 = pltpu.async_copy(cw_hbm.at[pl.ds(base, C)], cw_v, sem)
        d2 = pltpu.async_copy(ch_hbm.at[pl.ds(base, C)], ch_v, sem)
        d3 = pltpu.async_copy(cd_hbm.at[pl.ds(base, C)], cd_v, sem)
        d1.wait()
        d2.wait()
        d3.wait()

        def index_body(j, carry2):
            s = j * LANES
            cw = cw_v[pl.ds(s, LANES)]
            ch = ch_v[pl.ds(s, LANES)]
            cd = cd_v[pl.ds(s, LANES)]
            aw, ah, ad = jnp.abs(cw), jnp.abs(ch), jnp.abs(cd)
            in_f = (aw < 1.0) & (ah < 1.0) & (ad < 1.0)
            in_big = (aw < 4.0) & (ah < 4.0) & (ad < 4.0)
            inv_ext = jnp.where(in_f, 1.0, 0.25)
            szm1_f = jnp.where(in_f, 127.0, 255.0)
            szm1_i = jnp.where(in_f, 127, 255)
            str_w = jnp.where(in_f, FG_RES, BG_RES)
            str_hw = jnp.where(in_f, FG_RES * FG_RES, BG_RES * BG_RES)
            vbase = jnp.where(in_f, 0, FG_SIZE)
            x0, x1, wx = _axis(cw, inv_ext, szm1_f, szm1_i)
            y0, y1, wy = _axis(ch, inv_ext, szm1_f, szm1_i)
            z0, z1, wz = _axis(cd, inv_ext, szm1_f, szm1_i)
            zb0 = vbase + z0 * str_hw
            zb1 = vbase + z1 * str_hw
            r00 = zb0 + y0 * str_w
            r01 = zb0 + y1 * str_w
            r10 = zb1 + y0 * str_w
            r11 = zb1 + y1 * str_w
            idx_v[pl.ds(0 * C + s, LANES)] = r00 + x0
            idx_v[pl.ds(1 * C + s, LANES)] = r00 + x1
            idx_v[pl.ds(2 * C + s, LANES)] = r01 + x0
            idx_v[pl.ds(3 * C + s, LANES)] = r01 + x1
            idx_v[pl.ds(4 * C + s, LANES)] = r10 + x0
            idx_v[pl.ds(5 * C + s, LANES)] = r10 + x1
            idx_v[pl.ds(6 * C + s, LANES)] = r11 + x0
            idx_v[pl.ds(7 * C + s, LANES)] = r11 + x1
            wx_v[pl.ds(s, LANES)] = wx
            wy_v[pl.ds(s, LANES)] = wy
            wz_v[pl.ds(s, LANES)] = wz
            m_v[pl.ds(s, LANES)] = jnp.where(in_big, 0.0, 1.0)
            return carry2

        lax.fori_loop(0, G, index_body, 0)

        pltpu.async_copy(tab_hbm.at[idx_v], g_v, sem).wait()

        def combine_body(j, carry2):
            s = j * LANES
            c000 = g_v[pl.ds(0 * C + s, LANES)]
            c001 = g_v[pl.ds(1 * C + s, LANES)]
            c010 = g_v[pl.ds(2 * C + s, LANES)]
            c011 = g_v[pl.ds(3 * C + s, LANES)]
            c100 = g_v[pl.ds(4 * C + s, LANES)]
            c101 = g_v[pl.ds(5 * C + s, LANES)]
            c110 = g_v[pl.ds(6 * C + s, LANES)]
            c111 = g_v[pl.ds(7 * C + s, LANES)]
            wx = wx_v[pl.ds(s, LANES)]
            wy = wy_v[pl.ds(s, LANES)]
            wz = wz_v[pl.ds(s, LANES)]
            m = m_v[pl.ds(s, LANES)]
            c00 = c000 * (1.0 - wx) + c001 * wx
            c01 = c010 * (1.0 - wx) + c011 * wx
            c10 = c100 * (1.0 - wx) + c101 * wx
            c11 = c110 * (1.0 - wx) + c111 * wx
            c0 = c00 * (1.0 - wy) + c01 * wy
            c1 = c10 * (1.0 - wy) + c11 * wy
            res = c0 * (1.0 - wz) + c1 * wz
            out_v[pl.ds(s, LANES)] = jnp.where(m > 0.5, 1.0, res)
            return carry2

        lax.fori_loop(0, G, combine_body, 0)

        pltpu.sync_copy(out_v, out_hbm.at[pl.ds(base, C)])
        return carry

    lax.fori_loop(0, nchunks_w, chunk_body, 0)


def kernel(x_i, fg_sdf, bg_sdf):
    cw = x_i[:, 2]
    ch = x_i[:, 1]
    cd = x_i[:, 0]
    tab = jnp.concatenate([fg_sdf.reshape(-1), bg_sdf.reshape(-1)])
    return _sdf_kernel(cw, ch, cd, tab)
